# Initial kernel scaffold; baseline (speedup 1.0000x reference)
import jax
import jax.numpy as jnp
from jax import lax
from jax.experimental import pallas as pl
from jax.experimental.pallas import tpu as pltpu
from jax.experimental.pallas import tpu_sc as plsc

N = 10000
D = 128
E = 320000

NC = 2
NS = 16
NW = NC * NS

EPW = E // NW
C = 80
STEPS = EPW // C
RPW = N // NS
ZCH = 125

_mesh = plsc.VectorSubcoreMesh(core_axis_name="c", subcore_axis_name="s")


def _seg_kernel(z_hbm, src_hbm, dst_hbm, zrow_hbm, zcnt_hbm, one_hbm,
                out_hbm, outcnt_hbm,
                idx_s, idx_d, rows_v, ones_v, tbuf_v, cbuf_v,
                acc_sh, cnt_sh, sem):
    c = lax.axis_index("c")
    s = lax.axis_index("s")
    wid = c * NS + s

    pltpu.sync_copy(zrow_hbm, tbuf_v)
    pltpu.sync_copy(zcnt_hbm, cbuf_v)
    pltpu.sync_copy(one_hbm, ones_v)
    r0 = s * RPW
    for i in range(RPW // ZCH):
        pltpu.sync_copy(tbuf_v, acc_sh.at[pl.ds(r0 + i * ZCH, ZCH)])
        pltpu.sync_copy(cbuf_v, cnt_sh.at[pl.ds(r0 + i * ZCH, ZCH)])
    plsc.subcore_barrier()

    base = wid * EPW

    def body(i, carry):
        off = base + i * C
        pltpu.sync_copy(src_hbm.at[pl.ds(off, C)], idx_s)
        pltpu.sync_copy(dst_hbm.at[pl.ds(off, C)], idx_d)
        pltpu.async_copy(z_hbm.at[idx_s], rows_v, sem).wait()
        pltpu.sync_copy(rows_v, acc_sh.at[idx_d], add=True)
        pltpu.sync_copy(ones_v, cnt_sh.at[idx_d], add=True)
        return carry

    lax.fori_loop(0, STEPS, body, 0)
    plsc.subcore_barrier()

    for i in range(RPW // ZCH):
        pltpu.sync_copy(acc_sh.at[pl.ds(r0 + i * ZCH, ZCH)], tbuf_v)
        pltpu.sync_copy(tbuf_v, out_hbm.at[c, pl.ds(r0 + i * ZCH, ZCH)])
        pltpu.sync_copy(cnt_sh.at[pl.ds(r0 + i * ZCH, ZCH)], cbuf_v)
        pltpu.sync_copy(cbuf_v, outcnt_hbm.at[c, pl.ds(r0 + i * ZCH, ZCH)])


_seg_call = pl.kernel(
    _seg_kernel,
    out_type=(
        jax.ShapeDtypeStruct((NC, N, D), jnp.float32),
        jax.ShapeDtypeStruct((NC, N, 8), jnp.float32),
    ),
    mesh=_mesh,
    scratch_types=[
        pltpu.VMEM((C,), jnp.int32),
        pltpu.VMEM((C,), jnp.int32),
        pltpu.VMEM((C, D), jnp.float32),
        pltpu.VMEM((C, 8), jnp.float32),
        pltpu.VMEM((ZCH, D), jnp.float32),
        pltpu.VMEM((ZCH, 8), jnp.float32),
        pltpu.VMEM_SHARED((N, D), jnp.float32),
        pltpu.VMEM_SHARED((N, 8), jnp.float32),
        pltpu.SemaphoreType.DMA,
    ],
)


RB = 1000


def _pre_kernel(x_ref, wl_ref, wr_ref, bl_ref, z_ref, r_ref):
    x = x_ref[...]
    z_ref[...] = jnp.dot(x, wl_ref[...].T, preferred_element_type=jnp.float32)
    r_ref[...] = (
        jnp.dot(x, wr_ref[...].T, preferred_element_type=jnp.float32)
        + bl_ref[...]
    )


def _mid_kernel(acc_ref, cnt_ref, r_ref, wl_ref, wr_ref, bl_ref,
                z_ref, r2_ref):
    total = acc_ref[0] + acc_ref[1]
    cnt = cnt_ref[..., 0:1] + cnt_ref[..., 1:2]
    mean = total / jnp.maximum(cnt, 1.0)
    h = jnp.maximum(mean + r_ref[...], 0.0)
    z_ref[...] = jnp.dot(h, wl_ref[...].T, preferred_element_type=jnp.float32)
    r2_ref[...] = (
        jnp.dot(h, wr_ref[...].T, preferred_element_type=jnp.float32)
        + bl_ref[...]
    )


def _post_kernel(acc_ref, cnt_ref, r_ref, out_ref):
    total = acc_ref[0] + acc_ref[1]
    cnt = cnt_ref[..., 0:1] + cnt_ref[..., 1:2]
    y = total / jnp.maximum(cnt, 1.0) + r_ref[...]
    norm = jnp.sqrt(jnp.sum(y * y, axis=1, keepdims=True))
    out_ref[...] = y / jnp.maximum(norm, 1e-12)


def _row_spec(block):
    return pl.BlockSpec(block, lambda i: (i, 0))


_full_w = pl.BlockSpec((D, D), lambda i: (0, 0))
_full_b = pl.BlockSpec((1, D), lambda i: (0, 0))

_pre_call = pl.pallas_call(
    _pre_kernel,
    grid=(N // RB,),
    in_specs=[_row_spec((RB, D)), _full_w, _full_w, _full_b],
    out_specs=[_row_spec((RB, D)), _row_spec((RB, D))],
    out_shape=[
        jax.ShapeDtypeStruct((N, D), jnp.float32),
        jax.ShapeDtypeStruct((N, D), jnp.float32),
    ],
)

_acc_spec = pl.BlockSpec((NC, RB, D), lambda i: (0, i, 0))
_cnt_spec = pl.BlockSpec((RB, NC), lambda i: (i, 0))

_mid_call = pl.pallas_call(
    _mid_kernel,
    grid=(N // RB,),
    in_specs=[_acc_spec, _cnt_spec, _row_spec((RB, D)), _full_w, _full_w,
              _full_b],
    out_specs=[_row_spec((RB, D)), _row_spec((RB, D))],
    out_shape=[
        jax.ShapeDtypeStruct((N, D), jnp.float32),
        jax.ShapeDtypeStruct((N, D), jnp.float32),
    ],
)

_post_call = pl.pallas_call(
    _post_kernel,
    grid=(N // RB,),
    in_specs=[_acc_spec, _cnt_spec, _row_spec((RB, D))],
    out_specs=_row_spec((RB, D)),
    out_shape=jax.ShapeDtypeStruct((N, D), jnp.float32),
)


@jax.jit
def kernel(x, edge_index_list, Wl0, bl0, Wr0, Wl1, bl1, Wr1):
    ei = edge_index_list.astype(jnp.int32)
    src0, dst0 = ei[0, 0], ei[0, 1]
    src1, dst1 = ei[1, 0], ei[1, 1]

    zrow = jnp.zeros((ZCH, D), jnp.float32)
    zcnt = jnp.zeros((ZCH, 8), jnp.float32)
    one = jnp.ones((C, 8), jnp.float32)

    bl0_2d = bl0.reshape(1, D)
    bl1_2d = bl1.reshape(1, D)

    z0, r0 = _pre_call(x, Wl0, Wr0, bl0_2d)
    acc0, cnt0 = _seg_call(z0, src0, dst0, zrow, zcnt, one)
    cnt0_t = cnt0[:, :, 0].T
    z1, r1 = _mid_call(acc0, cnt0_t, r0, Wl1, Wr1, bl1_2d)
    acc1, cnt1 = _seg_call(z1, src1, dst1, zrow, zcnt, one)
    cnt1_t = cnt1[:, :, 0].T
    return _post_call(acc1, cnt1_t, r1)


# speedup vs baseline: 5.0820x; 5.0820x over previous
import jax
import jax.numpy as jnp
from jax import lax
from jax.experimental import pallas as pl
from jax.experimental.pallas import tpu as pltpu
from jax.experimental.pallas import tpu_sc as plsc

N = 10000
D = 128
E = 320000

NC = 2
NS = 16
NW = NC * NS

EPW = E // NW
C = 80
STEPS = EPW // C
NP = 10240
RPW = NP // NS
ZCH = 128

_mesh = plsc.VectorSubcoreMesh(core_axis_name="c", subcore_axis_name="s")


def _seg_kernel(z_hbm, src_hbm, dst_hbm, zrow_hbm, zcnt_hbm, one_hbm,
                out_hbm, outcnt_hbm,
                idx_s, idx_d, rows_v, ones_v, tbuf_v, cbuf_v,
                acc_sh, cnt_sh, sem):
    c = lax.axis_index("c")
    s = lax.axis_index("s")
    wid = c * NS + s

    pltpu.sync_copy(zrow_hbm, tbuf_v)
    pltpu.sync_copy(zcnt_hbm, cbuf_v)
    pltpu.sync_copy(one_hbm, ones_v)
    r0 = s * RPW
    for i in range(RPW // ZCH):
        pltpu.sync_copy(tbuf_v, acc_sh.at[pl.ds(r0 + i * ZCH, ZCH)])
    pltpu.sync_copy(cbuf_v, cnt_sh.at[pl.ds(r0, RPW)])
    plsc.subcore_barrier()

    base = wid * EPW

    def body(i, carry):
        off = base + i * C
        pltpu.sync_copy(src_hbm.at[pl.ds(off, C)], idx_s)
        pltpu.sync_copy(dst_hbm.at[pl.ds(off, C)], idx_d)
        pltpu.async_copy(z_hbm.at[idx_s], rows_v, sem).wait()
        pltpu.sync_copy(rows_v, acc_sh.at[idx_d], add=True)
        pltpu.sync_copy(ones_v, cnt_sh.at[idx_d], add=True)
        return carry

    lax.fori_loop(0, STEPS, body, 0)
    plsc.subcore_barrier()

    for i in range(RPW // ZCH):
        pltpu.sync_copy(acc_sh.at[pl.ds(r0 + i * ZCH, ZCH)], tbuf_v)
        pltpu.sync_copy(tbuf_v, out_hbm.at[c, pl.ds(r0 + i * ZCH, ZCH)])
    pltpu.sync_copy(cnt_sh.at[pl.ds(r0, RPW)], cbuf_v)
    pltpu.sync_copy(cbuf_v, outcnt_hbm.at[c, pl.ds(r0, RPW)])


_seg_call = pl.kernel(
    _seg_kernel,
    out_type=(
        jax.ShapeDtypeStruct((NC, NP, D), jnp.float32),
        jax.ShapeDtypeStruct((NC, NP), jnp.float32),
    ),
    mesh=_mesh,
    scratch_types=[
        pltpu.VMEM((C,), jnp.int32),
        pltpu.VMEM((C,), jnp.int32),
        pltpu.VMEM((C, D), jnp.float32),
        pltpu.VMEM((C,), jnp.float32),
        pltpu.VMEM((ZCH, D), jnp.float32),
        pltpu.VMEM((RPW,), jnp.float32),
        pltpu.VMEM_SHARED((NP, D), jnp.float32),
        pltpu.VMEM_SHARED((NP,), jnp.float32),
        pltpu.SemaphoreType.DMA,
    ],
)


RB = 1000


def _pre_kernel(x_ref, wl_ref, wr_ref, bl_ref, z_ref, r_ref):
    x = x_ref[...]
    z_ref[...] = jnp.dot(x, wl_ref[...].T, preferred_element_type=jnp.float32)
    r_ref[...] = (
        jnp.dot(x, wr_ref[...].T, preferred_element_type=jnp.float32)
        + bl_ref[...]
    )


def _mid_kernel(acc_ref, cnt_ref, r_ref, wl_ref, wr_ref, bl_ref,
                z_ref, r2_ref):
    total = acc_ref[0] + acc_ref[1]
    cnt = cnt_ref[..., 0:1] + cnt_ref[..., 1:2]
    mean = total / jnp.maximum(cnt, 1.0)
    h = jnp.maximum(mean + r_ref[...], 0.0)
    z_ref[...] = jnp.dot(h, wl_ref[...].T, preferred_element_type=jnp.float32)
    r2_ref[...] = (
        jnp.dot(h, wr_ref[...].T, preferred_element_type=jnp.float32)
        + bl_ref[...]
    )


def _post_kernel(acc_ref, cnt_ref, r_ref, out_ref):
    total = acc_ref[0] + acc_ref[1]
    cnt = cnt_ref[..., 0:1] + cnt_ref[..., 1:2]
    y = total / jnp.maximum(cnt, 1.0) + r_ref[...]
    norm = jnp.sqrt(jnp.sum(y * y, axis=1, keepdims=True))
    out_ref[...] = y / jnp.maximum(norm, 1e-12)


def _row_spec(block):
    return pl.BlockSpec(block, lambda i: (i, 0))


_full_w = pl.BlockSpec((D, D), lambda i: (0, 0))
_full_b = pl.BlockSpec((1, D), lambda i: (0, 0))

_pre_call = pl.pallas_call(
    _pre_kernel,
    grid=(N // RB,),
    in_specs=[_row_spec((RB, D)), _full_w, _full_w, _full_b],
    out_specs=[_row_spec((RB, D)), _row_spec((RB, D))],
    out_shape=[
        jax.ShapeDtypeStruct((N, D), jnp.float32),
        jax.ShapeDtypeStruct((N, D), jnp.float32),
    ],
)

_acc_spec = pl.BlockSpec((NC, RB, D), lambda i: (0, i, 0))
_cnt_spec = pl.BlockSpec((RB, NC), lambda i: (i, 0))

_mid_call = pl.pallas_call(
    _mid_kernel,
    grid=(N // RB,),
    in_specs=[_acc_spec, _cnt_spec, _row_spec((RB, D)), _full_w, _full_w,
              _full_b],
    out_specs=[_row_spec((RB, D)), _row_spec((RB, D))],
    out_shape=[
        jax.ShapeDtypeStruct((N, D), jnp.float32),
        jax.ShapeDtypeStruct((N, D), jnp.float32),
    ],
)

_post_call = pl.pallas_call(
    _post_kernel,
    grid=(N // RB,),
    in_specs=[_acc_spec, _cnt_spec, _row_spec((RB, D))],
    out_specs=_row_spec((RB, D)),
    out_shape=jax.ShapeDtypeStruct((N, D), jnp.float32),
)


@jax.jit
def kernel(x, edge_index_list, Wl0, bl0, Wr0, Wl1, bl1, Wr1):
    ei = edge_index_list.astype(jnp.int32)
    src0, dst0 = ei[0, 0], ei[0, 1]
    src1, dst1 = ei[1, 0], ei[1, 1]

    zrow = jnp.zeros((ZCH, D), jnp.float32)
    zcnt = jnp.zeros((RPW,), jnp.float32)
    one = jnp.ones((C,), jnp.float32)

    bl0_2d = bl0.reshape(1, D)
    bl1_2d = bl1.reshape(1, D)

    z0, r0 = _pre_call(x, Wl0, Wr0, bl0_2d)
    acc0, cnt0 = _seg_call(z0, src0, dst0, zrow, zcnt, one)
    z1, r1 = _mid_call(acc0[:, :N], cnt0[:, :N].T, r0, Wl1, Wr1, bl1_2d)
    acc1, cnt1 = _seg_call(z1, src1, dst1, zrow, zcnt, one)
    return _post_call(acc1[:, :N], cnt1[:, :N].T, r1)


# speedup vs baseline: 11.6091x; 2.2844x over previous
import jax
import jax.numpy as jnp
from jax import lax
from jax.experimental import pallas as pl
from jax.experimental.pallas import tpu as pltpu
from jax.experimental.pallas import tpu_sc as plsc

N = 10000
D = 128
E = 320000

NC = 2
NS = 16
NW = NC * NS

EPW = E // NW
C = 125
CPAD = 128
STEPS = EPW // C
NBUF = 2
OUTER = STEPS // NBUF
NP = 10240
RPW = NP // NS
ZCH = 80

_mesh = plsc.VectorSubcoreMesh(core_axis_name="c", subcore_axis_name="s")


def _seg_kernel(z_hbm, sdb_hbm, zrow_hbm,
                out_hbm, outcnt_hbm,
                idx_sd, rows_v, ones_v, cbuf_v,
                acc_sh, cnt_sh,
                semr0, semr1, semi0, semi1, semi2, semi3,
                semsc0, semsc1, semo0, semo1, semo2, semo3):
    semr = [semr0, semr1]
    semi = [semi0, semi1, semi2, semi3]
    semsc = [semsc0, semsc1]
    semo = [semo0, semo1, semo2, semo3]
    c = lax.axis_index("c")
    s = lax.axis_index("s")
    wid = c * NS + s

    ones16 = jnp.full((16,), 1.0, jnp.float32)
    zero16 = jnp.zeros((16,), jnp.float32)
    for k in range(CPAD // 16):
        ones_v[pl.ds(k * 16, 16)] = ones16
    for k in range(RPW // 16):
        cbuf_v[pl.ds(k * 16, 16)] = zero16

    pltpu.sync_copy(zrow_hbm, rows_v.at[0, pl.ds(0, ZCH)])
    r0 = s * RPW
    for i in range(RPW // ZCH):
        pltpu.async_copy(rows_v.at[0, pl.ds(0, ZCH)],
                         acc_sh.at[pl.ds(r0 + i * ZCH, ZCH)], semr[1])
    pltpu.async_copy(cbuf_v, cnt_sh.at[pl.ds(r0, RPW)], semr[0])
    for i in range(RPW // ZCH):
        pltpu.make_async_copy(rows_v.at[0, pl.ds(0, ZCH)],
                              acc_sh.at[pl.ds(r0 + i * ZCH, ZCH)],
                              semr[1]).wait()
    pltpu.make_async_copy(cbuf_v, cnt_sh.at[pl.ds(r0, RPW)], semr[0]).wait()
    plsc.subcore_barrier()

    def idx_start(j, t):
        pltpu.async_copy(sdb_hbm.at[wid, j], idx_sd.at[t], semi[t])

    def idx_wait(j, t):
        pltpu.make_async_copy(sdb_hbm.at[wid, j], idx_sd.at[t],
                              semi[t]).wait()

    def gather_start(b, t):
        pltpu.async_copy(z_hbm.at[idx_sd.at[t, 0]], rows_v.at[b], semr[b])

    def gather_wait(b, t):
        pltpu.make_async_copy(
            z_hbm.at[idx_sd.at[t, 0]], rows_v.at[b], semr[b]).wait()

    def rows_sc_start(b, t):
        pltpu.async_copy(rows_v.at[b], acc_sh.at[idx_sd.at[t, 1]], semsc[b],
                         add=True)

    def rows_sc_wait(b, t):
        pltpu.make_async_copy(rows_v.at[b], acc_sh.at[idx_sd.at[t, 1]],
                              semsc[b]).wait()

    def ones_sc_start(t):
        pltpu.async_copy(ones_v.at[pl.ds(0, C)], cnt_sh.at[idx_sd.at[t, 1]],
                         semo[t], add=True)

    def ones_sc_wait(t):
        pltpu.make_async_copy(ones_v.at[pl.ds(0, C)],
                              cnt_sh.at[idx_sd.at[t, 1]], semo[t]).wait()

    idx_start(0, 0)
    idx_start(1, 1)
    idx_wait(0, 0)
    gather_start(0, 0)

    def body(g, carry):
        for u in range(4):
            j = g * 4 + u
            b = u % 2
            gather_wait(b, u)
            rows_sc_start(b, u)
            ones_sc_start(u)

            @pl.when(j >= 1)
            def _():
                rows_sc_wait(1 - b, (u + 3) % 4)

            @pl.when(j + 1 < STEPS)
            def _():
                idx_wait(j + 1, (u + 1) % 4)
                gather_start(1 - b, (u + 1) % 4)

            @pl.when(j >= 2)
            def _():
                ones_sc_wait((u + 2) % 4)

            @pl.when(j + 2 < STEPS)
            def _():
                idx_start(j + 2, (u + 2) % 4)
        return carry

    lax.fori_loop(0, STEPS // 4, body, 0)
    rows_sc_wait((STEPS - 1) % 2, (STEPS - 1) % 4)
    ones_sc_wait((STEPS - 2) % 4)
    ones_sc_wait((STEPS - 1) % 4)
    plsc.subcore_barrier()

    pltpu.async_copy(acc_sh.at[pl.ds(r0, RPW)],
                     out_hbm.at[c, pl.ds(r0, RPW)], semr[0])
    pltpu.async_copy(cnt_sh.at[pl.ds(r0, RPW)],
                     outcnt_hbm.at[c, pl.ds(r0, RPW)], semr[1])
    pltpu.make_async_copy(acc_sh.at[pl.ds(r0, RPW)],
                          out_hbm.at[c, pl.ds(r0, RPW)], semr[0]).wait()
    pltpu.make_async_copy(cnt_sh.at[pl.ds(r0, RPW)],
                          outcnt_hbm.at[c, pl.ds(r0, RPW)], semr[1]).wait()


_seg_call = pl.kernel(
    _seg_kernel,
    out_type=(
        jax.ShapeDtypeStruct((NC, NP, D), jnp.float32),
        jax.ShapeDtypeStruct((NC, NP), jnp.float32),
    ),
    mesh=_mesh,
    scratch_types=[
        pltpu.VMEM((4, 2, C), jnp.int32),
        pltpu.VMEM((NBUF, C, D), jnp.float32),
        pltpu.VMEM((CPAD,), jnp.float32),
        pltpu.VMEM((RPW,), jnp.float32),
        pltpu.VMEM_SHARED((NP, D), jnp.float32),
        pltpu.VMEM_SHARED((NP,), jnp.float32),
    ] + [pltpu.SemaphoreType.DMA] * 12,
)


RB = 1000


def _mid_kernel(acc_ref, cnt_ref, x_ref, wl_ref, wr_ref, bl_ref, h_ref):
    total = acc_ref[0] + acc_ref[1]
    cnt = jnp.sum(cnt_ref[...], axis=1, keepdims=True)
    mean = total / jnp.maximum(cnt, 1.0)
    lin = (
        jnp.dot(mean, wl_ref[...].T, preferred_element_type=jnp.float32)
        + jnp.dot(x_ref[...], wr_ref[...].T,
                  preferred_element_type=jnp.float32)
        + bl_ref[...]
    )
    h_ref[...] = jnp.maximum(lin, 0.0)


def _post_kernel(acc_ref, cnt_ref, h_ref, wl_ref, wr_ref, bl_ref, out_ref):
    total = acc_ref[0] + acc_ref[1]
    cnt = jnp.sum(cnt_ref[...], axis=1, keepdims=True)
    mean = total / jnp.maximum(cnt, 1.0)
    y = (
        jnp.dot(mean, wl_ref[...].T, preferred_element_type=jnp.float32)
        + jnp.dot(h_ref[...], wr_ref[...].T,
                  preferred_element_type=jnp.float32)
        + bl_ref[...]
    )
    norm = jnp.sqrt(jnp.sum(y * y, axis=1, keepdims=True))
    out_ref[...] = y / jnp.maximum(norm, 1e-12)


def _row_spec(block):
    return pl.BlockSpec(block, lambda i: (i, 0))


_full_w = pl.BlockSpec((D, D), lambda i: (0, 0))
_full_b = pl.BlockSpec((1, D), lambda i: (0, 0))
_acc_spec = pl.BlockSpec((NC, RB, D), lambda i: (0, i, 0))
_cnt_spec = pl.BlockSpec((RB, NC), lambda i: (i, 0))

_mid_call = pl.pallas_call(
    _mid_kernel,
    grid=(N // RB,),
    in_specs=[_acc_spec, _cnt_spec, _row_spec((RB, D)), _full_w, _full_w,
              _full_b],
    out_specs=_row_spec((RB, D)),
    out_shape=jax.ShapeDtypeStruct((N, D), jnp.float32),
)

_post_call = pl.pallas_call(
    _post_kernel,
    grid=(N // RB,),
    in_specs=[_acc_spec, _cnt_spec, _row_spec((RB, D)), _full_w, _full_w,
              _full_b],
    out_specs=_row_spec((RB, D)),
    out_shape=jax.ShapeDtypeStruct((N, D), jnp.float32),
)


@jax.jit
def kernel(x, edge_index_list, Wl0, bl0, Wr0, Wl1, bl1, Wr1):
    ei = edge_index_list.astype(jnp.int32)
    sdb0 = ei[0].reshape(2, NW, STEPS, C).transpose(1, 2, 0, 3)
    sdb1 = ei[1].reshape(2, NW, STEPS, C).transpose(1, 2, 0, 3)

    zrow = jnp.zeros((ZCH, D), jnp.float32)

    bl0_2d = bl0.reshape(1, D)
    bl1_2d = bl1.reshape(1, D)

    acc0, cnt0 = _seg_call(x, sdb0, zrow)
    h = _mid_call(acc0, cnt0.T, x, Wl0, Wr0, bl0_2d)
    acc1, cnt1 = _seg_call(h, sdb1, zrow)
    return _post_call(acc1, cnt1.T, h, Wl1, Wr1, bl1_2d)


# speedup vs baseline: 14.0232x; 1.2079x over previous
import jax
import jax.numpy as jnp
from jax import lax
from jax.experimental import pallas as pl
from jax.experimental.pallas import tpu as pltpu
from jax.experimental.pallas import tpu_sc as plsc

N = 10000
D = 128
E = 320000

NC = 2
NS = 16
NW = NC * NS

EPW = E // NW
C = 80
CPAD = 80
STEPS = EPW // C
NRB = 3
NIB = 6
UNR = 6
MAIN = (STEPS // UNR) * UNR
NP = 10240
RPW = NP // NS
ZCH = 80

_mesh = plsc.VectorSubcoreMesh(core_axis_name="c", subcore_axis_name="s")


def _seg_kernel(z_hbm, sdb_hbm, zrow_hbm,
                out_hbm, outcnt_hbm,
                idx_sd, rows_v, ones_v, cbuf_v,
                acc_sh, cnt_sh,
                semr0, semr1, semr2,
                semi0, semi1, semi2, semi3, semi4, semi5,
                semsc0, semsc1, semsc2,
                semo0, semo1, semo2, semo3, semo4, semo5):
    semr = [semr0, semr1, semr2]
    semi = [semi0, semi1, semi2, semi3, semi4, semi5]
    semsc = [semsc0, semsc1, semsc2]
    semo = [semo0, semo1, semo2, semo3, semo4, semo5]
    c = lax.axis_index("c")
    s = lax.axis_index("s")
    wid = c * NS + s

    ones16 = jnp.full((16,), 1.0, jnp.float32)
    zero16 = jnp.zeros((16,), jnp.float32)
    for k in range(CPAD // 16):
        ones_v[pl.ds(k * 16, 16)] = ones16
    for k in range(RPW // 16):
        cbuf_v[pl.ds(k * 16, 16)] = zero16

    pltpu.sync_copy(zrow_hbm, rows_v.at[0, pl.ds(0, ZCH)])
    r0 = s * RPW
    for i in range(RPW // ZCH):
        pltpu.async_copy(rows_v.at[0, pl.ds(0, ZCH)],
                         acc_sh.at[pl.ds(r0 + i * ZCH, ZCH)], semr[1])
    pltpu.async_copy(cbuf_v, cnt_sh.at[pl.ds(r0, RPW)], semr[0])
    for i in range(RPW // ZCH):
        pltpu.make_async_copy(rows_v.at[0, pl.ds(0, ZCH)],
                              acc_sh.at[pl.ds(r0 + i * ZCH, ZCH)],
                              semr[1]).wait()
    pltpu.make_async_copy(cbuf_v, cnt_sh.at[pl.ds(r0, RPW)], semr[0]).wait()
    plsc.subcore_barrier()

    def idx_start(j, t):
        pltpu.async_copy(sdb_hbm.at[wid, j], idx_sd.at[t], semi[t])

    def idx_wait(j, t):
        pltpu.make_async_copy(sdb_hbm.at[wid, j], idx_sd.at[t],
                              semi[t]).wait()

    def gather_start(b, t):
        pltpu.async_copy(z_hbm.at[idx_sd.at[t, 0]], rows_v.at[b], semr[b])

    def gather_wait(b, t):
        pltpu.make_async_copy(
            z_hbm.at[idx_sd.at[t, 0]], rows_v.at[b], semr[b]).wait()

    def rows_sc_start(b, t):
        pltpu.async_copy(rows_v.at[b], acc_sh.at[idx_sd.at[t, 1]], semsc[b],
                         add=True)

    def rows_sc_wait(b, t):
        pltpu.make_async_copy(rows_v.at[b], acc_sh.at[idx_sd.at[t, 1]],
                              semsc[b]).wait()

    def ones_sc_start(t):
        pltpu.async_copy(ones_v.at[pl.ds(0, C)], cnt_sh.at[idx_sd.at[t, 1]],
                         semo[t], add=True)

    def ones_sc_wait(t):
        pltpu.make_async_copy(ones_v.at[pl.ds(0, C)],
                              cnt_sh.at[idx_sd.at[t, 1]], semo[t]).wait()

    for t in range(5):
        idx_start(t, t)
    idx_wait(0, 0)
    gather_start(0, 0)
    idx_wait(1, 1)
    gather_start(1, 1)

    def body(g, carry):
        for u in range(UNR):
            j = g * UNR + u

            @pl.when(j >= 1)
            def _():
                rows_sc_wait((u + 2) % NRB, (u + 5) % NIB)
                ones_sc_wait((u + 5) % NIB)

            gather_wait(u % NRB, u)
            rows_sc_start(u % NRB, u)
            ones_sc_start(u)

            @pl.when(j + 2 < STEPS)
            def _():
                idx_wait(j + 2, (u + 2) % NIB)
                gather_start((u + 2) % NRB, (u + 2) % NIB)

            @pl.when(j + 5 < STEPS)
            def _():
                idx_start(j + 5, (u + 5) % NIB)
        return carry

    lax.fori_loop(0, MAIN // UNR, body, 0)
    for j in range(MAIN, STEPS):
        s3 = j % NRB
        s6 = j % NIB
        rows_sc_wait((s3 + 2) % NRB, (s6 + 5) % NIB)
        ones_sc_wait((s6 + 5) % NIB)
        gather_wait(s3, s6)
        rows_sc_start(s3, s6)
        ones_sc_start(s6)
        if j + 2 < STEPS:
            idx_wait(j + 2, (s6 + 2) % NIB)
            gather_start((s3 + 2) % NRB, (s6 + 2) % NIB)
        if j + 5 < STEPS:
            idx_start(j + 5, (s6 + 5) % NIB)
    rows_sc_wait((STEPS - 1) % NRB, (STEPS - 1) % NIB)
    ones_sc_wait((STEPS - 1) % NIB)
    plsc.subcore_barrier()

    pltpu.async_copy(acc_sh.at[pl.ds(r0, RPW)],
                     out_hbm.at[c, pl.ds(r0, RPW)], semr[0])
    pltpu.async_copy(cnt_sh.at[pl.ds(r0, RPW)],
                     outcnt_hbm.at[c, pl.ds(r0, RPW)], semr[1])
    pltpu.make_async_copy(acc_sh.at[pl.ds(r0, RPW)],
                          out_hbm.at[c, pl.ds(r0, RPW)], semr[0]).wait()
    pltpu.make_async_copy(cnt_sh.at[pl.ds(r0, RPW)],
                          outcnt_hbm.at[c, pl.ds(r0, RPW)], semr[1]).wait()


_seg_call = pl.kernel(
    _seg_kernel,
    out_type=(
        jax.ShapeDtypeStruct((NC, NP, D), jnp.float32),
        jax.ShapeDtypeStruct((NC, NP), jnp.float32),
    ),
    mesh=_mesh,
    scratch_types=[
        pltpu.VMEM((NIB, 2, C), jnp.int32),
        pltpu.VMEM((NRB, C, D), jnp.float32),
        pltpu.VMEM((CPAD,), jnp.float32),
        pltpu.VMEM((RPW,), jnp.float32),
        pltpu.VMEM_SHARED((NP, D), jnp.float32),
        pltpu.VMEM_SHARED((NP,), jnp.float32),
    ] + [pltpu.SemaphoreType.DMA] * 18,
)


RB = 1000


def _mid_kernel(acc_ref, cnt_ref, x_ref, wl_ref, wr_ref, bl_ref, h_ref):
    total = acc_ref[0] + acc_ref[1]
    cnt = jnp.sum(cnt_ref[...], axis=1, keepdims=True)
    mean = total / jnp.maximum(cnt, 1.0)
    lin = (
        jnp.dot(mean, wl_ref[...].T, preferred_element_type=jnp.float32)
        + jnp.dot(x_ref[...], wr_ref[...].T,
                  preferred_element_type=jnp.float32)
        + bl_ref[...]
    )
    h_ref[...] = jnp.maximum(lin, 0.0)


def _post_kernel(acc_ref, cnt_ref, h_ref, wl_ref, wr_ref, bl_ref, out_ref):
    total = acc_ref[0] + acc_ref[1]
    cnt = jnp.sum(cnt_ref[...], axis=1, keepdims=True)
    mean = total / jnp.maximum(cnt, 1.0)
    y = (
        jnp.dot(mean, wl_ref[...].T, preferred_element_type=jnp.float32)
        + jnp.dot(h_ref[...], wr_ref[...].T,
                  preferred_element_type=jnp.float32)
        + bl_ref[...]
    )
    norm = jnp.sqrt(jnp.sum(y * y, axis=1, keepdims=True))
    out_ref[...] = y / jnp.maximum(norm, 1e-12)


def _row_spec(block):
    return pl.BlockSpec(block, lambda i: (i, 0))


_full_w = pl.BlockSpec((D, D), lambda i: (0, 0))
_full_b = pl.BlockSpec((1, D), lambda i: (0, 0))
_acc_spec = pl.BlockSpec((NC, RB, D), lambda i: (0, i, 0))
_cnt_spec = pl.BlockSpec((RB, NC), lambda i: (i, 0))

_mid_call = pl.pallas_call(
    _mid_kernel,
    grid=(N // RB,),
    in_specs=[_acc_spec, _cnt_spec, _row_spec((RB, D)), _full_w, _full_w,
              _full_b],
    out_specs=_row_spec((RB, D)),
    out_shape=jax.ShapeDtypeStruct((N, D), jnp.float32),
)

_post_call = pl.pallas_call(
    _post_kernel,
    grid=(N // RB,),
    in_specs=[_acc_spec, _cnt_spec, _row_spec((RB, D)), _full_w, _full_w,
              _full_b],
    out_specs=_row_spec((RB, D)),
    out_shape=jax.ShapeDtypeStruct((N, D), jnp.float32),
)


@jax.jit
def kernel(x, edge_index_list, Wl0, bl0, Wr0, Wl1, bl1, Wr1):
    ei = edge_index_list.astype(jnp.int32)
    sdb0 = ei[0].reshape(2, NW, STEPS, C).transpose(1, 2, 0, 3)
    sdb1 = ei[1].reshape(2, NW, STEPS, C).transpose(1, 2, 0, 3)

    zrow = jnp.zeros((ZCH, D), jnp.float32)

    bl0_2d = bl0.reshape(1, D)
    bl1_2d = bl1.reshape(1, D)

    acc0, cnt0 = _seg_call(x, sdb0, zrow)
    h = _mid_call(acc0, cnt0.T, x, Wl0, Wr0, bl0_2d)
    acc1, cnt1 = _seg_call(h, sdb1, zrow)
    return _post_call(acc1, cnt1.T, h, Wl1, Wr1, bl1_2d)


# speedup vs baseline: 14.2960x; 1.0195x over previous
import jax
import jax.numpy as jnp
from jax import lax
from jax.experimental import pallas as pl
from jax.experimental.pallas import tpu as pltpu
from jax.experimental.pallas import tpu_sc as plsc

N = 10000
D = 128
E = 320000

NC = 2
NS = 16
NW = NC * NS

EPW = E // NW
C = 80
CPAD = 80
STEPS = EPW // C
NRB = 3
NIB = 6
UNR = 6
MAIN = (STEPS // UNR) * UNR
NP = 10240
RPW = NP // NS
ZCH = 80

_mesh = plsc.VectorSubcoreMesh(core_axis_name="c", subcore_axis_name="s")


def _seg_kernel(z_hbm, sdb_hbm, zrow_hbm,
                out_hbm, outcnt_hbm,
                idx_sd, rows_v, ones_v, cbuf_v,
                acc_sh, cnt_sh,
                semr0, semr1, semr2,
                semi0, semi1, semi2, semi3, semi4, semi5,
                semsc0, semsc1, semsc2,
                semo0, semo1, semo2, semo3, semo4, semo5):
    semr = [semr0, semr1, semr2]
    semi = [semi0, semi1, semi2, semi3, semi4, semi5]
    semsc = [semsc0, semsc1, semsc2]
    semo = [semo0, semo1, semo2, semo3, semo4, semo5]
    c = lax.axis_index("c")
    s = lax.axis_index("s")
    wid = c * NS + s

    ones16 = jnp.full((16,), 1.0, jnp.float32)
    zero16 = jnp.zeros((16,), jnp.float32)
    for k in range(CPAD // 16):
        ones_v[pl.ds(k * 16, 16)] = ones16
    for k in range(RPW // 16):
        cbuf_v[pl.ds(k * 16, 16)] = zero16

    pltpu.sync_copy(zrow_hbm, rows_v.at[0, pl.ds(0, ZCH)])
    r0 = s * RPW
    for i in range(RPW // ZCH):
        pltpu.async_copy(rows_v.at[0, pl.ds(0, ZCH)],
                         acc_sh.at[pl.ds(r0 + i * ZCH, ZCH)], semr[1])
    pltpu.async_copy(cbuf_v, cnt_sh.at[pl.ds(r0, RPW)], semr[0])
    for i in range(RPW // ZCH):
        pltpu.make_async_copy(rows_v.at[0, pl.ds(0, ZCH)],
                              acc_sh.at[pl.ds(r0 + i * ZCH, ZCH)],
                              semr[1]).wait()
    pltpu.make_async_copy(cbuf_v, cnt_sh.at[pl.ds(r0, RPW)], semr[0]).wait()
    plsc.subcore_barrier()

    def idx_start(j, t):
        pltpu.async_copy(sdb_hbm.at[wid, j], idx_sd.at[t], semi[t])

    def idx_wait(j, t):
        pltpu.make_async_copy(sdb_hbm.at[wid, j], idx_sd.at[t],
                              semi[t]).wait()

    def gather_start(b, t):
        pltpu.async_copy(z_hbm.at[idx_sd.at[t, 0]], rows_v.at[b], semr[b])

    def gather_wait(b, t):
        pltpu.make_async_copy(
            z_hbm.at[idx_sd.at[t, 0]], rows_v.at[b], semr[b]).wait()

    def rows_sc_start(b, t):
        pltpu.async_copy(rows_v.at[b], acc_sh.at[idx_sd.at[t, 1]], semsc[b],
                         add=True)

    def rows_sc_wait(b, t):
        pltpu.make_async_copy(rows_v.at[b], acc_sh.at[idx_sd.at[t, 1]],
                              semsc[b]).wait()

    def ones_sc_start(t):
        pltpu.async_copy(ones_v.at[pl.ds(0, C)], cnt_sh.at[idx_sd.at[t, 1]],
                         semo[t], add=True)

    def ones_sc_wait(t):
        pltpu.make_async_copy(ones_v.at[pl.ds(0, C)],
                              cnt_sh.at[idx_sd.at[t, 1]], semo[t]).wait()

    for t in range(5):
        idx_start(t, t)
    idx_wait(0, 0)
    gather_start(0, 0)
    idx_wait(1, 1)
    gather_start(1, 1)

    def body(g, carry):
        for u in range(UNR):
            j = g * UNR + u

            @pl.when(j >= 1)
            def _():
                rows_sc_wait((u + 2) % NRB, (u + 5) % NIB)
                ones_sc_wait((u + 5) % NIB)

            gather_wait(u % NRB, u)
            rows_sc_start(u % NRB, u)
            ones_sc_start(u)

            @pl.when(j + 2 < STEPS)
            def _():
                idx_wait(j + 2, (u + 2) % NIB)
                gather_start((u + 2) % NRB, (u + 2) % NIB)

            @pl.when(j + 5 < STEPS)
            def _():
                idx_start(j + 5, (u + 5) % NIB)
        return carry

    lax.fori_loop(0, MAIN // UNR, body, 0)
    for j in range(MAIN, STEPS):
        s3 = j % NRB
        s6 = j % NIB
        rows_sc_wait((s3 + 2) % NRB, (s6 + 5) % NIB)
        ones_sc_wait((s6 + 5) % NIB)
        gather_wait(s3, s6)
        rows_sc_start(s3, s6)
        ones_sc_start(s6)
        if j + 2 < STEPS:
            idx_wait(j + 2, (s6 + 2) % NIB)
            gather_start((s3 + 2) % NRB, (s6 + 2) % NIB)
        if j + 5 < STEPS:
            idx_start(j + 5, (s6 + 5) % NIB)
    rows_sc_wait((STEPS - 1) % NRB, (STEPS - 1) % NIB)
    ones_sc_wait((STEPS - 1) % NIB)
    plsc.subcore_barrier()

    pltpu.async_copy(acc_sh.at[pl.ds(r0, RPW)],
                     out_hbm.at[c, pl.ds(r0, RPW)], semr[0])
    pltpu.async_copy(cnt_sh.at[pl.ds(r0, RPW)],
                     outcnt_hbm.at[c, pl.ds(r0, RPW)], semr[1])
    pltpu.make_async_copy(acc_sh.at[pl.ds(r0, RPW)],
                          out_hbm.at[c, pl.ds(r0, RPW)], semr[0]).wait()
    pltpu.make_async_copy(cnt_sh.at[pl.ds(r0, RPW)],
                          outcnt_hbm.at[c, pl.ds(r0, RPW)], semr[1]).wait()


_seg_call = pl.kernel(
    _seg_kernel,
    out_type=(
        jax.ShapeDtypeStruct((NC, NP, D), jnp.float32),
        jax.ShapeDtypeStruct((NC, NP), jnp.float32),
    ),
    mesh=_mesh,
    scratch_types=[
        pltpu.VMEM((NIB, 2, C), jnp.int32),
        pltpu.VMEM((NRB, C, D), jnp.float32),
        pltpu.VMEM((CPAD,), jnp.float32),
        pltpu.VMEM((RPW,), jnp.float32),
        pltpu.VMEM_SHARED((NP, D), jnp.float32),
        pltpu.VMEM_SHARED((NP,), jnp.float32),
    ] + [pltpu.SemaphoreType.DMA] * 18,
)


RB = 2000


def _mid_kernel(acc_ref, cnt_ref, x_ref, wl_ref, wr_ref, bl_ref, h_ref):
    total = acc_ref[0] + acc_ref[1]
    cnt = jnp.sum(cnt_ref[...], axis=1, keepdims=True)
    mean = total / jnp.maximum(cnt, 1.0)
    lin = (
        jnp.dot(mean, wl_ref[...].T, preferred_element_type=jnp.float32)
        + jnp.dot(x_ref[...], wr_ref[...].T,
                  preferred_element_type=jnp.float32)
        + bl_ref[...]
    )
    h_ref[...] = jnp.maximum(lin, 0.0)


def _post_kernel(acc_ref, cnt_ref, h_ref, wl_ref, wr_ref, bl_ref, out_ref):
    total = acc_ref[0] + acc_ref[1]
    cnt = jnp.sum(cnt_ref[...], axis=1, keepdims=True)
    mean = total / jnp.maximum(cnt, 1.0)
    y = (
        jnp.dot(mean, wl_ref[...].T, preferred_element_type=jnp.float32)
        + jnp.dot(h_ref[...], wr_ref[...].T,
                  preferred_element_type=jnp.float32)
        + bl_ref[...]
    )
    norm = jnp.sqrt(jnp.sum(y * y, axis=1, keepdims=True))
    out_ref[...] = y / jnp.maximum(norm, 1e-12)


def _row_spec(block):
    return pl.BlockSpec(block, lambda i: (i, 0))


_full_w = pl.BlockSpec((D, D), lambda i: (0, 0))
_full_b = pl.BlockSpec((1, D), lambda i: (0, 0))
_acc_spec = pl.BlockSpec((NC, RB, D), lambda i: (0, i, 0))
_cnt_spec = pl.BlockSpec((RB, NC), lambda i: (i, 0))

_mid_call = pl.pallas_call(
    _mid_kernel,
    grid=(N // RB,),
    in_specs=[_acc_spec, _cnt_spec, _row_spec((RB, D)), _full_w, _full_w,
              _full_b],
    out_specs=_row_spec((RB, D)),
    out_shape=jax.ShapeDtypeStruct((N, D), jnp.float32),
)

_post_call = pl.pallas_call(
    _post_kernel,
    grid=(N // RB,),
    in_specs=[_acc_spec, _cnt_spec, _row_spec((RB, D)), _full_w, _full_w,
              _full_b],
    out_specs=_row_spec((RB, D)),
    out_shape=jax.ShapeDtypeStruct((N, D), jnp.float32),
)


@jax.jit
def kernel(x, edge_index_list, Wl0, bl0, Wr0, Wl1, bl1, Wr1):
    ei = edge_index_list.astype(jnp.int32)
    sdb0 = ei[0].reshape(2, NW, STEPS, C).transpose(1, 2, 0, 3)
    sdb1 = ei[1].reshape(2, NW, STEPS, C).transpose(1, 2, 0, 3)

    zrow = jnp.zeros((ZCH, D), jnp.float32)

    bl0_2d = bl0.reshape(1, D)
    bl1_2d = bl1.reshape(1, D)

    acc0, cnt0 = _seg_call(x, sdb0, zrow)
    h = _mid_call(acc0, cnt0.T, x, Wl0, Wr0, bl0_2d)
    acc1, cnt1 = _seg_call(h, sdb1, zrow)
    return _post_call(acc1, cnt1.T, h, Wl1, Wr1, bl1_2d)


# speedup vs baseline: 14.3611x; 1.0046x over previous
import jax
import jax.numpy as jnp
from jax import lax
from jax.experimental import pallas as pl
from jax.experimental.pallas import tpu as pltpu
from jax.experimental.pallas import tpu_sc as plsc

N = 10000
D = 128
E = 320000

NC = 2
NS = 16
NW = NC * NS

EPW = E // NW
C = 80
CPAD = 80
STEPS = EPW // C
NRB = 3
NIB = 6
UNR = 6
MAIN = (STEPS // UNR) * UNR
NP = 10240
RPW = NP // NS
ZCH = 80

_mesh = plsc.VectorSubcoreMesh(core_axis_name="c", subcore_axis_name="s")


def _seg_kernel(z_hbm, sdb_hbm, zrow_hbm,
                out_hbm, outcnt_hbm,
                idx_sd, rows_v, ones_v, cbuf_v,
                acc_sh, cnt_sh,
                semr0, semr1, semr2,
                semi0, semi1, semi2, semi3, semi4, semi5,
                semsc0, semsc1, semsc2,
                semo0, semo1, semo2, semo3, semo4, semo5):
    semr = [semr0, semr1, semr2]
    semi = [semi0, semi1, semi2, semi3, semi4, semi5]
    semsc = [semsc0, semsc1, semsc2]
    semo = [semo0, semo1, semo2, semo3, semo4, semo5]
    c = lax.axis_index("c")
    s = lax.axis_index("s")
    wid = c * NS + s

    ones16 = jnp.full((16,), 1.0, jnp.float32)
    zero16 = jnp.zeros((16,), jnp.float32)
    for k in range(CPAD // 16):
        ones_v[pl.ds(k * 16, 16)] = ones16
    for k in range(RPW // 16):
        cbuf_v[pl.ds(k * 16, 16)] = zero16

    def idx_start(j, t):
        pltpu.async_copy(sdb_hbm.at[wid, j], idx_sd.at[t], semi[t])

    def idx_wait(j, t):
        pltpu.make_async_copy(sdb_hbm.at[wid, j], idx_sd.at[t],
                              semi[t]).wait()

    def gather_start(b, t):
        pltpu.async_copy(z_hbm.at[idx_sd.at[t, 0]], rows_v.at[b], semr[b])

    def gather_wait(b, t):
        pltpu.make_async_copy(
            z_hbm.at[idx_sd.at[t, 0]], rows_v.at[b], semr[b]).wait()

    def rows_sc_start(b, t):
        pltpu.async_copy(rows_v.at[b], acc_sh.at[idx_sd.at[t, 1]], semsc[b],
                         add=True)

    def rows_sc_wait(b, t):
        pltpu.make_async_copy(rows_v.at[b], acc_sh.at[idx_sd.at[t, 1]],
                              semsc[b]).wait()

    def ones_sc_start(t):
        pltpu.async_copy(ones_v.at[pl.ds(0, C)], cnt_sh.at[idx_sd.at[t, 1]],
                         semo[t], add=True)

    def ones_sc_wait(t):
        pltpu.make_async_copy(ones_v.at[pl.ds(0, C)],
                              cnt_sh.at[idx_sd.at[t, 1]], semo[t]).wait()

    for t in range(5):
        idx_start(t, t)
    idx_wait(0, 0)
    gather_start(0, 0)
    idx_wait(1, 1)
    gather_start(1, 1)

    pltpu.sync_copy(zrow_hbm, rows_v.at[2, pl.ds(0, ZCH)])
    r0 = s * RPW
    for i in range(RPW // ZCH):
        pltpu.async_copy(rows_v.at[2, pl.ds(0, ZCH)],
                         acc_sh.at[pl.ds(r0 + i * ZCH, ZCH)], semsc[0])
    pltpu.async_copy(cbuf_v, cnt_sh.at[pl.ds(r0, RPW)], semsc[1])
    for i in range(RPW // ZCH):
        pltpu.make_async_copy(rows_v.at[2, pl.ds(0, ZCH)],
                              acc_sh.at[pl.ds(r0 + i * ZCH, ZCH)],
                              semsc[0]).wait()
    pltpu.make_async_copy(cbuf_v, cnt_sh.at[pl.ds(r0, RPW)],
                          semsc[1]).wait()
    plsc.subcore_barrier()

    def body(g, carry):
        for u in range(UNR):
            j = g * UNR + u

            @pl.when(j >= 1)
            def _():
                rows_sc_wait((u + 2) % NRB, (u + 5) % NIB)
                ones_sc_wait((u + 5) % NIB)

            gather_wait(u % NRB, u)
            rows_sc_start(u % NRB, u)
            ones_sc_start(u)

            @pl.when(j + 2 < STEPS)
            def _():
                idx_wait(j + 2, (u + 2) % NIB)
                gather_start((u + 2) % NRB, (u + 2) % NIB)

            @pl.when(j + 5 < STEPS)
            def _():
                idx_start(j + 5, (u + 5) % NIB)
        return carry

    lax.fori_loop(0, MAIN // UNR, body, 0)
    for j in range(MAIN, STEPS):
        s3 = j % NRB
        s6 = j % NIB
        rows_sc_wait((s3 + 2) % NRB, (s6 + 5) % NIB)
        ones_sc_wait((s6 + 5) % NIB)
        gather_wait(s3, s6)
        rows_sc_start(s3, s6)
        ones_sc_start(s6)
        if j + 2 < STEPS:
            idx_wait(j + 2, (s6 + 2) % NIB)
            gather_start((s3 + 2) % NRB, (s6 + 2) % NIB)
        if j + 5 < STEPS:
            idx_start(j + 5, (s6 + 5) % NIB)
    rows_sc_wait((STEPS - 1) % NRB, (STEPS - 1) % NIB)
    ones_sc_wait((STEPS - 1) % NIB)
    plsc.subcore_barrier()

    pltpu.async_copy(acc_sh.at[pl.ds(r0, RPW)],
                     out_hbm.at[c, pl.ds(r0, RPW)], semr[0])
    pltpu.async_copy(cnt_sh.at[pl.ds(r0, RPW)],
                     outcnt_hbm.at[c, pl.ds(r0, RPW)], semr[1])
    pltpu.make_async_copy(acc_sh.at[pl.ds(r0, RPW)],
                          out_hbm.at[c, pl.ds(r0, RPW)], semr[0]).wait()
    pltpu.make_async_copy(cnt_sh.at[pl.ds(r0, RPW)],
                          outcnt_hbm.at[c, pl.ds(r0, RPW)], semr[1]).wait()


_seg_call = pl.kernel(
    _seg_kernel,
    out_type=(
        jax.ShapeDtypeStruct((NC, NP, D), jnp.float32),
        jax.ShapeDtypeStruct((NC, NP), jnp.float32),
    ),
    mesh=_mesh,
    scratch_types=[
        pltpu.VMEM((NIB, 2, C), jnp.int32),
        pltpu.VMEM((NRB, C, D), jnp.float32),
        pltpu.VMEM((CPAD,), jnp.float32),
        pltpu.VMEM((RPW,), jnp.float32),
        pltpu.VMEM_SHARED((NP, D), jnp.float32),
        pltpu.VMEM_SHARED((NP,), jnp.float32),
    ] + [pltpu.SemaphoreType.DMA] * 18,
)


RB = 2000


def _mid_kernel(acc_ref, cnt_ref, x_ref, wl_ref, wr_ref, bl_ref, h_ref):
    total = acc_ref[0] + acc_ref[1]
    cnt = jnp.sum(cnt_ref[...], axis=1, keepdims=True)
    mean = total / jnp.maximum(cnt, 1.0)
    lin = (
        jnp.dot(mean, wl_ref[...].T, preferred_element_type=jnp.float32)
        + jnp.dot(x_ref[...], wr_ref[...].T,
                  preferred_element_type=jnp.float32)
        + bl_ref[...]
    )
    h_ref[...] = jnp.maximum(lin, 0.0)


def _post_kernel(acc_ref, cnt_ref, h_ref, wl_ref, wr_ref, bl_ref, out_ref):
    total = acc_ref[0] + acc_ref[1]
    cnt = jnp.sum(cnt_ref[...], axis=1, keepdims=True)
    mean = total / jnp.maximum(cnt, 1.0)
    y = (
        jnp.dot(mean, wl_ref[...].T, preferred_element_type=jnp.float32)
        + jnp.dot(h_ref[...], wr_ref[...].T,
                  preferred_element_type=jnp.float32)
        + bl_ref[...]
    )
    norm = jnp.sqrt(jnp.sum(y * y, axis=1, keepdims=True))
    out_ref[...] = y / jnp.maximum(norm, 1e-12)


def _row_spec(block):
    return pl.BlockSpec(block, lambda i: (i, 0))


_full_w = pl.BlockSpec((D, D), lambda i: (0, 0))
_full_b = pl.BlockSpec((1, D), lambda i: (0, 0))
_acc_spec = pl.BlockSpec((NC, RB, D), lambda i: (0, i, 0))
_cnt_spec = pl.BlockSpec((RB, NC), lambda i: (i, 0))

_mid_call = pl.pallas_call(
    _mid_kernel,
    grid=(N // RB,),
    in_specs=[_acc_spec, _cnt_spec, _row_spec((RB, D)), _full_w, _full_w,
              _full_b],
    out_specs=_row_spec((RB, D)),
    out_shape=jax.ShapeDtypeStruct((N, D), jnp.float32),
)

_post_call = pl.pallas_call(
    _post_kernel,
    grid=(N // RB,),
    in_specs=[_acc_spec, _cnt_spec, _row_spec((RB, D)), _full_w, _full_w,
              _full_b],
    out_specs=_row_spec((RB, D)),
    out_shape=jax.ShapeDtypeStruct((N, D), jnp.float32),
)


@jax.jit
def kernel(x, edge_index_list, Wl0, bl0, Wr0, Wl1, bl1, Wr1):
    ei = edge_index_list.astype(jnp.int32)
    sdb0 = ei[0].reshape(2, NW, STEPS, C).transpose(1, 2, 0, 3)
    sdb1 = ei[1].reshape(2, NW, STEPS, C).transpose(1, 2, 0, 3)

    zrow = jnp.zeros((ZCH, D), jnp.float32)

    bl0_2d = bl0.reshape(1, D)
    bl1_2d = bl1.reshape(1, D)

    acc0, cnt0 = _seg_call(x, sdb0, zrow)
    h = _mid_call(acc0, cnt0.T, x, Wl0, Wr0, bl0_2d)
    acc1, cnt1 = _seg_call(h, sdb1, zrow)
    return _post_call(acc1, cnt1.T, h, Wl1, Wr1, bl1_2d)
